# baseline (speedup 1.0000x reference)
import functools

import jax
import jax.numpy as jnp
from jax import lax
from jax.experimental import pallas as pl
from jax.experimental.pallas import tpu as pltpu
from jax.experimental.pallas import tpu_sc as plsc

R = 224
K = 512
D = 64 * (R // 16) * (R // 16)
N = 8
D_CHUNK = 1792
N_CHUNKS = D // D_CHUNK


def _vq_dist_body(z_ref, w_ref, j_ref, loss_ref, score_ref, zsq_ref):
    i = pl.program_id(0)

    @pl.when(i == 0)
    def _init():
        score_ref[...] = jnp.zeros_like(score_ref)
        zsq_ref[...] = jnp.zeros_like(zsq_ref)

    z = z_ref[...]
    w = w_ref[...]
    g = lax.dot_general(
        z, w, (((1,), (1,)), ((), ())),
        precision=lax.Precision.HIGHEST,
        preferred_element_type=jnp.float32,
    )
    wsq = jnp.sum(w * w, axis=1)
    score_ref[...] += wsq[None, :] - 2.0 * g
    zsq_ref[:, 0:1] += jnp.sum(z * z, axis=1, keepdims=True)

    @pl.when(i == N_CHUNKS - 1)
    def _finish():
        score = score_ref[...]
        minv = jnp.min(score, axis=1, keepdims=True)
        idx = lax.broadcasted_iota(jnp.int32, (N, K), 1)
        j = jnp.min(jnp.where(score == minv, idx, K), axis=1, keepdims=True)
        j_ref[...] = jnp.broadcast_to(j, (N, 128))
        zsq = zsq_ref[:, 0:1]
        loss_ref[...] = (jnp.sum(zsq + minv) / N).reshape(1, 1)


def _vq_dist(z, w):
    return pl.pallas_call(
        _vq_dist_body,
        grid=(N_CHUNKS,),
        in_specs=[
            pl.BlockSpec((N, D_CHUNK), lambda i: (0, i)),
            pl.BlockSpec((K, D_CHUNK), lambda i: (0, i)),
        ],
        out_specs=[
            pl.BlockSpec((N, 128), lambda i: (0, 0)),
            pl.BlockSpec((1, 1), lambda i: (0, 0)),
        ],
        out_shape=[
            jax.ShapeDtypeStruct((N, 128), jnp.int32),
            jax.ShapeDtypeStruct((1, 1), jnp.float32),
        ],
        scratch_shapes=[
            pltpu.VMEM((N, K), jnp.float32),
            pltpu.VMEM((N, 128), jnp.float32),
        ],
    )(z, w)


def _sc_gather(w, j):
    mesh = plsc.VectorSubcoreMesh(core_axis_name="c", subcore_axis_name="s")

    @functools.partial(
        pl.kernel,
        mesh=mesh,
        out_type=jax.ShapeDtypeStruct((N, D), jnp.float32),
        scratch_types=[
            pltpu.VMEM((N,), jnp.int32),
            pltpu.VMEM((N, D), jnp.float32),
            pltpu.SemaphoreType.DMA,
        ],
    )
    def gather_k(w_hbm, idx_hbm, out_hbm, idx_v, rows_v, sem):
        wid = lax.axis_index("s") * 2 + lax.axis_index("c")

        @pl.when(wid == 0)
        def _():
            pltpu.sync_copy(idx_hbm, idx_v)
            pltpu.async_copy(w_hbm.at[idx_v], rows_v, sem).wait()
            pltpu.sync_copy(rows_v, out_hbm)

    return gather_k(w, j)


def _conv2d(x, w, b, stride, pad):
    out = lax.conv_general_dilated(
        x, jnp.transpose(w, (2, 3, 1, 0)), (stride, stride),
        ((pad, pad), (pad, pad)),
        dimension_numbers=("NHWC", "HWIO", "NHWC"),
        precision=lax.Precision.HIGH)
    return out + b[None, None, None, :]


def _bn(x, g, b, eps=1e-5):
    m = jnp.mean(x, axis=(0, 1, 2), keepdims=True)
    v = jnp.mean(x * x, axis=(0, 1, 2), keepdims=True) - m * m
    scale = g[None, None, None, :] / jnp.sqrt(v + eps)
    return x * scale + (b[None, None, None, :] - m * scale)


def _lrelu(x):
    return jnp.where(x >= 0, x, 0.01 * x)


def _up(x):
    return jnp.repeat(jnp.repeat(x, 2, axis=1), 2, axis=2)


def _s2d_conv(x, w, b):
    n, hh, ww, c = x.shape
    o = w.shape[0]
    xb = x.reshape(n, hh // 2, 2, ww // 2, 2, c)
    xb = jnp.transpose(xb, (0, 1, 3, 2, 4, 5)).reshape(n, hh // 2, ww // 2, 4 * c)
    wp = jnp.pad(w, ((0, 0), (0, 0), (1, 1), (1, 1)))
    k = wp.reshape(o, c, 3, 2, 3, 2)
    k = jnp.transpose(k, (2, 4, 3, 5, 1, 0)).reshape(3, 3, 4 * c, o)
    y = lax.conv_general_dilated(
        xb, k, (1, 1), ((1, 1), (1, 1)),
        dimension_numbers=("NHWC", "HWIO", "NHWC"))
    return y + b[None, None, None, :]


def _encoder(x, p):
    x = _lrelu(_bn(_conv2d(x, p['w_e1'], p['b_e1'], 2, 1), p['g_e1'], p['bb_e1']))
    x = _lrelu(_bn(_conv2d(x, p['w_e2'], p['b_e2'], 2, 1), p['g_e2'], p['bb_e2']))
    x = _lrelu(_bn(_conv2d(x, p['w_e3'], p['b_e3'], 2, 1), p['g_e3'], p['bb_e3']))
    x = _lrelu(_bn(_conv2d(x, p['w_e4'], p['b_e4'], 2, 1), p['g_e4'], p['bb_e4']))
    x = _lrelu(_bn(_conv2d(x, p['w_e5'], p['b_e5'], 1, 0), p['g_e5'], p['bb_e5']))
    return x


def _decoder(z, p):
    x = z
    x = _lrelu(_bn(_conv2d(x, p['w_d0'], p['b_d0'], 1, 0), p['g_d0'], p['bb_d0']))
    x = _dec_layer(x, p['w_d1'], p['b_d1'], p['g_d1'], p['bb_d1'])
    x = _dec_layer(x, p['w_d2'], p['b_d2'], p['g_d2'], p['bb_d2'])
    x = _dec_layer(x, p['w_d3'], p['b_d3'], p['g_d3'], p['bb_d3'])
    x = _shuffle(jax.nn.sigmoid(_up_conv_packed(x, p['w_d4'], p['b_d4'])), 3)
    return jnp.transpose(x, (0, 3, 1, 2)).reshape(-1, 3, R, R)


def _phase_kernel(w):
    o_, i_, _, _ = w.shape
    z = jnp.zeros_like(w[:, :, 0:1, :])
    ra0 = jnp.concatenate([w[:, :, 0:1], w[:, :, 1:2] + w[:, :, 2:3], z], axis=2)
    ra1 = jnp.concatenate([z, w[:, :, 0:1] + w[:, :, 1:2], w[:, :, 2:3]], axis=2)

    def cols(wr):
        zc = jnp.zeros_like(wr[:, :, :, 0:1])
        c0 = jnp.concatenate([wr[:, :, :, 0:1], wr[:, :, :, 1:2] + wr[:, :, :, 2:3], zc], axis=3)
        c1 = jnp.concatenate([zc, wr[:, :, :, 0:1] + wr[:, :, :, 1:2], wr[:, :, :, 2:3]], axis=3)
        return c0, c1

    k00, k01 = cols(ra0)
    k10, k11 = cols(ra1)
    k = jnp.stack([k00, k01, k10, k11], axis=0)
    k = jnp.transpose(k, (3, 4, 2, 0, 1)).reshape(3, 3, i_, 4 * o_)
    return k


def _up_conv_packed(x, w, b):
    o_ = w.shape[0]
    y = lax.conv_general_dilated(
        x.astype(jnp.bfloat16), _phase_kernel(w).astype(jnp.bfloat16),
        (1, 1), ((1, 1), (1, 1)),
        dimension_numbers=("NHWC", "HWIO", "NHWC"),
        preferred_element_type=jnp.float32)
    return y + jnp.tile(b, 4)[None, None, None, :]


def _shuffle(y, o_):
    n, h, w_sp, _ = y.shape
    y = y.reshape(n, h, w_sp, 2, 2, o_)
    return jnp.transpose(y, (0, 1, 3, 2, 4, 5)).reshape(n, 2 * h, 2 * w_sp, o_)


def _dec_layer(x, w, b, g, bb, eps=1e-5):
    o_ = w.shape[0]
    yp = _up_conv_packed(x, w, b)
    m4 = jnp.mean(yp, axis=(0, 1, 2))
    e2 = jnp.mean(yp * yp, axis=(0, 1, 2))
    m = jnp.mean(m4.reshape(4, o_), axis=0)
    v = jnp.mean(e2.reshape(4, o_), axis=0) - m * m
    scale = g / jnp.sqrt(v + eps)
    shift = bb - m * scale
    s4 = jnp.tile(scale, 4)[None, None, None, :]
    t4 = jnp.tile(shift, 4)[None, None, None, :]
    yn = yp * s4 + t4
    yn = _lrelu(yn).astype(jnp.bfloat16)
    return _shuffle(yn, o_)


def kernel(x, params):
    xh = jnp.transpose(x, (0, 2, 3, 1))
    ze = _encoder(xh, params)
    n, h, w_, c = ze.shape
    z = ze.reshape(-1, D)
    wdict = params['dict']

    j_out, loss = _vq_dist(z, wdict)
    j = j_out[:, 0]

    wj = _sc_gather(wdict, j)

    ze2 = wj.reshape(n, h, w_, c)
    recon = _decoder(ze2, params)
    loss_s = loss[0, 0]
    return recon, loss_s, loss_s


# speedup vs baseline: 1.4232x; 1.4232x over previous
import functools

import jax
import jax.numpy as jnp
from jax import lax
from jax.experimental import pallas as pl
from jax.experimental.pallas import tpu as pltpu
from jax.experimental.pallas import tpu_sc as plsc

R = 224
K = 512
D = 64 * (R // 16) * (R // 16)
N = 8
D_CHUNK = 1792
N_CHUNKS = D // D_CHUNK


def _vq_dist_body(z_ref, w_ref, j_ref, loss_ref, score_ref, zsq_ref):
    i = pl.program_id(0)

    @pl.when(i == 0)
    def _init():
        score_ref[...] = jnp.zeros_like(score_ref)
        zsq_ref[...] = jnp.zeros_like(zsq_ref)

    z = z_ref[...]
    w = w_ref[...]
    g = lax.dot_general(
        z, w, (((1,), (1,)), ((), ())),
        precision=lax.Precision.HIGHEST,
        preferred_element_type=jnp.float32,
    )
    wsq = jnp.sum(w * w, axis=1)
    score_ref[...] += wsq[None, :] - 2.0 * g
    zsq_ref[:, 0:1] += jnp.sum(z * z, axis=1, keepdims=True)

    @pl.when(i == N_CHUNKS - 1)
    def _finish():
        score = score_ref[...]
        minv = jnp.min(score, axis=1, keepdims=True)
        idx = lax.broadcasted_iota(jnp.int32, (N, K), 1)
        j = jnp.min(jnp.where(score == minv, idx, K), axis=1, keepdims=True)
        j_ref[...] = jnp.broadcast_to(j, (N, 128))
        zsq = zsq_ref[:, 0:1]
        loss_ref[...] = (jnp.sum(zsq + minv) / N).reshape(1, 1)


def _vq_dist(z, w):
    return pl.pallas_call(
        _vq_dist_body,
        grid=(N_CHUNKS,),
        in_specs=[
            pl.BlockSpec((N, D_CHUNK), lambda i: (0, i)),
            pl.BlockSpec((K, D_CHUNK), lambda i: (0, i)),
        ],
        out_specs=[
            pl.BlockSpec((N, 128), lambda i: (0, 0)),
            pl.BlockSpec((1, 1), lambda i: (0, 0)),
        ],
        out_shape=[
            jax.ShapeDtypeStruct((N, 128), jnp.int32),
            jax.ShapeDtypeStruct((1, 1), jnp.float32),
        ],
        scratch_shapes=[
            pltpu.VMEM((N, K), jnp.float32),
            pltpu.VMEM((N, 128), jnp.float32),
        ],
    )(z, w)


def _sc_gather(w, j):
    mesh = plsc.VectorSubcoreMesh(core_axis_name="c", subcore_axis_name="s")

    @functools.partial(
        pl.kernel,
        mesh=mesh,
        out_type=jax.ShapeDtypeStruct((N, D), jnp.float32),
        scratch_types=[
            pltpu.VMEM((N,), jnp.int32),
            pltpu.VMEM((N, D), jnp.float32),
            pltpu.SemaphoreType.DMA,
        ],
    )
    def gather_k(w_hbm, idx_hbm, out_hbm, idx_v, rows_v, sem):
        wid = lax.axis_index("s") * 2 + lax.axis_index("c")

        @pl.when(wid == 0)
        def _():
            pltpu.sync_copy(idx_hbm, idx_v)
            pltpu.async_copy(w_hbm.at[idx_v], rows_v, sem).wait()
            pltpu.sync_copy(rows_v, out_hbm)

    return gather_k(w, j)


def _conv2d(x, w, b, stride, pad):
    out = lax.conv_general_dilated(
        x, jnp.transpose(w, (2, 3, 1, 0)), (stride, stride),
        ((pad, pad), (pad, pad)),
        dimension_numbers=("NHWC", "HWIO", "NHWC"))
    return out + b[None, None, None, :]


def _bn(x, g, b, eps=1e-5):
    m = jnp.mean(x, axis=(0, 1, 2), keepdims=True)
    v = jnp.mean(x * x, axis=(0, 1, 2), keepdims=True) - m * m
    scale = g[None, None, None, :] / jnp.sqrt(v + eps)
    return x * scale + (b[None, None, None, :] - m * scale)


def _lrelu(x):
    return jnp.where(x >= 0, x, 0.01 * x)


def _up(x):
    return jnp.repeat(jnp.repeat(x, 2, axis=1), 2, axis=2)


def _s2d_conv(x, w, b):
    n, hh, ww, c = x.shape
    o = w.shape[0]
    xb = x.reshape(n, hh // 2, 2, ww // 2, 2, c)
    xb = jnp.transpose(xb, (0, 1, 3, 2, 4, 5)).reshape(n, hh // 2, ww // 2, 4 * c)
    wp = jnp.pad(w, ((0, 0), (0, 0), (1, 1), (1, 1)))
    k = wp.reshape(o, c, 3, 2, 3, 2)
    k = jnp.transpose(k, (2, 4, 3, 5, 1, 0)).reshape(3, 3, 4 * c, o)
    y = lax.conv_general_dilated(
        xb, k, (1, 1), ((1, 1), (1, 1)),
        dimension_numbers=("NHWC", "HWIO", "NHWC"))
    return y + b[None, None, None, :]


def _encoder(x, p):
    x = _lrelu(_bn(_conv2d(x, p['w_e1'], p['b_e1'], 2, 1), p['g_e1'], p['bb_e1']))
    x = _lrelu(_bn(_conv2d(x, p['w_e2'], p['b_e2'], 2, 1), p['g_e2'], p['bb_e2']))
    x = _lrelu(_bn(_conv2d(x, p['w_e3'], p['b_e3'], 2, 1), p['g_e3'], p['bb_e3']))
    x = _lrelu(_bn(_conv2d(x, p['w_e4'], p['b_e4'], 2, 1), p['g_e4'], p['bb_e4']))
    x = _lrelu(_bn(_conv2d(x, p['w_e5'], p['b_e5'], 1, 0), p['g_e5'], p['bb_e5']))
    return x


def _decoder(z, p):
    x = z
    x = _lrelu(_bn(_conv2d(x, p['w_d0'], p['b_d0'], 1, 0), p['g_d0'], p['bb_d0']))
    x = _dec_layer(x, p['w_d1'], p['b_d1'], p['g_d1'], p['bb_d1'])
    x = _dec_layer(x, p['w_d2'], p['b_d2'], p['g_d2'], p['bb_d2'])
    x = _dec_layer(x, p['w_d3'], p['b_d3'], p['g_d3'], p['bb_d3'])
    y = jax.nn.sigmoid(_up_conv_packed(x, p['w_d4'], p['b_d4']))
    n, hb, wb, _ = y.shape
    y = y.reshape(n, hb, wb, 2, 2, 3)
    return jnp.transpose(y, (0, 5, 1, 3, 2, 4)).reshape(-1, 3, R, R)


def _phase_kernel(w):
    o_, i_, _, _ = w.shape
    z = jnp.zeros_like(w[:, :, 0:1, :])
    ra0 = jnp.concatenate([w[:, :, 0:1], w[:, :, 1:2] + w[:, :, 2:3], z], axis=2)
    ra1 = jnp.concatenate([z, w[:, :, 0:1] + w[:, :, 1:2], w[:, :, 2:3]], axis=2)

    def cols(wr):
        zc = jnp.zeros_like(wr[:, :, :, 0:1])
        c0 = jnp.concatenate([wr[:, :, :, 0:1], wr[:, :, :, 1:2] + wr[:, :, :, 2:3], zc], axis=3)
        c1 = jnp.concatenate([zc, wr[:, :, :, 0:1] + wr[:, :, :, 1:2], wr[:, :, :, 2:3]], axis=3)
        return c0, c1

    k00, k01 = cols(ra0)
    k10, k11 = cols(ra1)
    k = jnp.stack([k00, k01, k10, k11], axis=0)
    k = jnp.transpose(k, (3, 4, 2, 0, 1)).reshape(3, 3, i_, 4 * o_)
    return k


def _up_conv_packed_nobias(x, w):
    y = lax.conv_general_dilated(
        x.astype(jnp.bfloat16), _phase_kernel(w).astype(jnp.bfloat16),
        (1, 1), ((1, 1), (1, 1)),
        dimension_numbers=("NHWC", "HWIO", "NHWC"),
        preferred_element_type=jnp.float32)
    return y


def _up_conv_packed(x, w, b):
    return _up_conv_packed_nobias(x, w) + jnp.tile(b, 4)[None, None, None, :]


def _shuffle(y, o_):
    n, h, w_sp, _ = y.shape
    y = y.reshape(n, h, w_sp, 2, 2, o_)
    return jnp.transpose(y, (0, 1, 3, 2, 4, 5)).reshape(n, 2 * h, 2 * w_sp, o_)


def _dec_layer(x, w, b, g, bb, eps=1e-5):
    o_ = w.shape[0]
    yp = _up_conv_packed_nobias(x, w)
    m4 = jnp.mean(yp, axis=(0, 1, 2))
    e2 = jnp.mean(yp * yp, axis=(0, 1, 2))
    m = jnp.mean(m4.reshape(4, o_), axis=0)
    v = jnp.mean(e2.reshape(4, o_), axis=0) - m * m
    scale = g / jnp.sqrt(v + eps)
    shift = bb - m * scale
    s4 = jnp.tile(scale, 4)[None, None, None, :]
    t4 = jnp.tile(shift, 4)[None, None, None, :]
    yn = yp * s4 + t4
    yn = _lrelu(yn).astype(jnp.bfloat16)
    return _shuffle(yn, o_)


def kernel(x, params):
    xh = jnp.transpose(x, (0, 2, 3, 1))
    ze = _encoder(xh, params)
    n, h, w_, c = ze.shape
    z = ze.reshape(-1, D)
    wdict = params['dict']

    j_out, loss = _vq_dist(z, wdict)
    j = j_out[:, 0]

    wj = _sc_gather(wdict, j)

    ze2 = wj.reshape(n, h, w_, c)
    recon = _decoder(ze2, params)
    loss_s = loss[0, 0]
    return recon, loss_s, loss_s
